# baseline (speedup 1.0000x reference)
import functools

import jax
import jax.numpy as jnp
import numpy as np
from jax import lax
from jax.experimental import pallas as pl
from jax.experimental.pallas import tpu as pltpu
from jax.experimental.pallas import tpu_sc as plsc

B, D, H, W = 8, 128, 128, 128
ROWS = B * D * H
NC, NS = 2, 16
NW = NC * NS
ROWS_W = ROWS // NW
CHUNK = 128
NCH = ROWS_W // CHUNK


def _source_rows() -> np.ndarray:
    key = jax.random.key(42)
    ka, kb = jax.random.split(key)
    apply_transform = jax.random.bernoulli(ka, 0.4, (B,))
    koefs = jnp.where(apply_transform, jax.random.randint(kb, (B,), 1, 4), 0)
    rot = np.asarray(koefs) > 0
    r = np.arange(ROWS)
    b = r >> 14
    d = (r >> 7) & (H - 1)
    h = r & (H - 1)
    src = np.where(rot[b], (b << 14) + ((H - 1 - h) << 7) + d, r)
    return src.astype(np.int32).reshape(ROWS // CHUNK, CHUNK)


_IDX = _source_rows()


NBUF = 4


def _build_sc_kernel():
    mesh = plsc.VectorSubcoreMesh(core_axis_name="c", subcore_axis_name="s")
    f32 = jnp.float32

    @functools.partial(
        pl.kernel,
        mesh=mesh,
        out_type=[jax.ShapeDtypeStruct((ROWS, W), f32)] * 3,
        scratch_types=[
            pltpu.VMEM((NCH, CHUNK), jnp.int32),
            pltpu.VMEM((NBUF, CHUNK, W), f32),
        ] + [pltpu.SemaphoreType.DMA] * NBUF,
    )
    def k(idx_hbm, v_in, m_in, s_in, v_out, m_out, s_out,
          idx_v, rows, *sems):
        wid = lax.axis_index("s") * NC + lax.axis_index("c")
        base = wid * ROWS_W
        pltpu.sync_copy(idx_hbm.at[pl.ds(wid * NCH, NCH)], idx_v)

        pairs = ((v_in, v_out), (m_in, m_out), (s_in, s_out))

        def start_gather(j, b, ih):
            pltpu.async_copy(ih.at[idx_v.at[j]], rows.at[b], sems[b])

        def wait_gather(b):
            pltpu.make_async_copy(v_in.at[idx_v.at[0]], rows.at[b],
                                  sems[b]).wait()

        def start_store(j, b, oh):
            pltpu.async_copy(rows.at[b],
                             oh.at[pl.ds(base + j * CHUNK, CHUNK)], sems[b])

        def wait_store(b):
            pltpu.make_async_copy(rows.at[b], v_out.at[pl.ds(base, CHUNK)],
                                  sems[b]).wait()

        def step(j, b, gather, oh, head=False):
            wait_gather(b)
            b2 = (b + 2) % NBUF
            if not head:
                wait_store(b2)
            if gather is not None:
                start_gather(gather[0], b2, gather[1])
            start_store(j, b, oh)

        start_gather(0, 0, v_in)
        start_gather(1, 1, v_in)
        for t, (ih, oh) in enumerate(pairs):
            nxt_ih = pairs[t + 1][0] if t + 1 < len(pairs) else None
            for i in range(NBUF):
                step(i, i, (i + 2, ih), oh, head=(t == 0 and i < 2))

            def round_body(r, carry, ih=ih, oh=oh):
                for i in range(NBUF):
                    j = NBUF * r + i
                    step(j, i, (j + 2, ih), oh)
                return carry

            lax.fori_loop(1, NCH // NBUF - 1, round_body, 0)

            for i in range(NBUF):
                j = NCH - NBUF + i
                if j + 2 < NCH:
                    gather = (j + 2, ih)
                elif nxt_ih is not None:
                    gather = (j + 2 - NCH, nxt_ih)
                else:
                    gather = None
                step(j, i, gather, oh)
        wait_store((NCH - 2) % NBUF)
        wait_store((NCH - 1) % NBUF)

    return k


_SC_KERNEL = _build_sc_kernel()


def kernel(volume, gt_mask, gt_skel):
    vi = volume.reshape(ROWS, W)
    mi = gt_mask.reshape(ROWS, W)
    si = gt_skel.reshape(ROWS, W)
    vo, mo, so = _SC_KERNEL(jnp.asarray(_IDX), vi, mi, si)
    shape = (B, D, H, W)
    return (vo.reshape(shape), mo.reshape(shape), so.reshape(shape))


# speedup vs baseline: 1.0056x; 1.0056x over previous
import functools

import jax
import jax.numpy as jnp
import numpy as np
from jax import lax
from jax.experimental import pallas as pl
from jax.experimental.pallas import tpu as pltpu
from jax.experimental.pallas import tpu_sc as plsc

B, D, H, W = 8, 128, 128, 128
ROWS = B * D * H
NC, NS = 2, 16
NW = NC * NS
ROWS_W = ROWS // NW
CHUNK = 128
NCH = ROWS_W // CHUNK


def _source_rows() -> np.ndarray:
    key = jax.random.key(42)
    ka, kb = jax.random.split(key)
    apply_transform = jax.random.bernoulli(ka, 0.4, (B,))
    koefs = jnp.where(apply_transform, jax.random.randint(kb, (B,), 1, 4), 0)
    rot = np.asarray(koefs) > 0
    r = np.arange(ROWS)
    b = r >> 14
    d = (r >> 7) & (H - 1)
    h = r & (H - 1)
    src = np.where(rot[b], (b << 14) + ((H - 1 - h) << 7) + d, r)
    return src.astype(np.int32).reshape(ROWS // CHUNK, CHUNK)


_IDX = _source_rows()


NBUF = 4


def _build_sc_kernel():
    mesh = plsc.VectorSubcoreMesh(core_axis_name="c", subcore_axis_name="s")
    f32 = jnp.float32

    @functools.partial(
        pl.kernel,
        mesh=mesh,
        out_type=[jax.ShapeDtypeStruct((ROWS, W), f32)] * 3,
        scratch_types=[
            pltpu.VMEM((NCH, CHUNK), jnp.int32),
            pltpu.VMEM((NBUF, CHUNK, W), f32),
        ] + [pltpu.SemaphoreType.DMA] * NBUF,
    )
    def k(idx_hbm, v_in, m_in, s_in, v_out, m_out, s_out,
          idx_v, rows, *sems):
        wid = lax.axis_index("s") * NC + lax.axis_index("c")
        base = wid * ROWS_W
        pltpu.sync_copy(idx_hbm.at[pl.ds(wid * NCH, NCH)], idx_v)

        for ih, oh in ((v_in, v_out), (m_in, m_out), (s_in, s_out)):
            def start_gather(j, b, ih=ih):
                pltpu.async_copy(ih.at[idx_v.at[j]], rows.at[b], sems[b])

            def wait_gather(b, ih=ih):
                pltpu.make_async_copy(ih.at[idx_v.at[0]], rows.at[b],
                                      sems[b]).wait()

            def start_store(j, b, oh=oh):
                pltpu.async_copy(rows.at[b],
                                 oh.at[pl.ds(base + j * CHUNK, CHUNK)],
                                 sems[b])

            def wait_store(b, oh=oh):
                pltpu.make_async_copy(rows.at[b], oh.at[pl.ds(base, CHUNK)],
                                      sems[b]).wait()

            def step(j, b, head, tail):
                wait_gather(b)
                b2 = (b + 2) % NBUF
                if not head:
                    wait_store(b2)
                if not tail:
                    start_gather(j + 2, b2)
                start_store(j, b)

            start_gather(0, 0)
            start_gather(1, 1)
            for i in range(NBUF):
                step(i, i, head=(i < 2), tail=False)

            def round_body(r, carry):
                for i in range(NBUF):
                    step(NBUF * r + i, i, head=False, tail=False)
                return carry

            lax.fori_loop(1, NCH // NBUF - 1, round_body, 0)

            for i in range(NBUF):
                j = NCH - NBUF + i
                step(j, i, head=False, tail=(j + 2 >= NCH))
            wait_store((NCH - 2) % NBUF)
            wait_store((NCH - 1) % NBUF)

    return k


_SC_KERNEL = _build_sc_kernel()


def kernel(volume, gt_mask, gt_skel):
    vi = volume.reshape(ROWS, W)
    mi = gt_mask.reshape(ROWS, W)
    si = gt_skel.reshape(ROWS, W)
    vo, mo, so = _SC_KERNEL(jnp.asarray(_IDX), vi, mi, si)
    shape = (B, D, H, W)
    return (vo.reshape(shape), mo.reshape(shape), so.reshape(shape))
